# baseline (speedup 1.0000x reference)
import jax
import jax.numpy as jnp
from jax.experimental import pallas as pl
from jax.experimental.pallas import tpu as pltpu

_ISSUE, _DEV, _FILE = 4000, 2000, 4000
_N = _ISSUE + _DEV + _FILE
_D = 128
_LEAKY = 0.1
_R = 400
_PREC = jax.lax.Precision.HIGHEST
_PREC_BIG = jax.lax.Precision.DEFAULT

_SEGS = ((0, _ISSUE), (_ISSUE, _DEV), (_ISSUE + _DEV, _FILE))


def _lk(x):
    return jnp.where(x >= 0, x, _LEAKY * x)


def _proj_body(ie, de, fe, ih, dh, fh, aa):
    aa[0:_ISSUE, :] = jnp.dot(ie[...], ih[...], precision=_PREC)
    aa[_ISSUE:_ISSUE + _DEV, :] = jnp.dot(de[...], dh[...], precision=_PREC)
    aa[_ISSUE + _DEV:_N, :] = jnp.dot(fe[...], fh[...], precision=_PREC)


def _proj(ie, de, fe, ih, dh, fh):
    return pl.pallas_call(
        _proj_body,
        out_shape=jax.ShapeDtypeStruct((_N, _D), jnp.float32),
    )(ie, de, fe, ih, dh, fh)


def _layer_body(adj, lat, aa, aa_blk, s_in, tem, hyp, latn, s_out, inner):
    i = pl.program_id(0)

    @pl.when(i == 0)
    def _prologue():
        for s, (st, sz) in enumerate(_SEGS):
            inner[s * _D:(s + 1) * _D, :] = _lk(jax.lax.dot_general(
                aa[st:st + sz, :], lat[st:st + sz, :],
                (((0,), (0,)), ((), ())), precision=_PREC))

    t = _lk(jnp.dot(adj[...], lat[...], precision=_PREC_BIG))

    b0, b1 = _ISSUE // _R, (_ISSUE + _DEV) // _R
    for s, lo, hi in ((0, 0, b0), (1, b0, b1), (2, b1, _N // _R)):
        @pl.when((i >= lo) & (i < hi))
        def _seg(s=s):
            hyp[...] = _lk(jnp.dot(
                aa_blk[...], inner[s * _D:(s + 1) * _D, :], precision=_PREC))

    ln = t + hyp[...]
    tem[...] = t
    latn[...] = ln
    s_out[...] = s_in[...] + ln


def _layer(adj, lat, aa, s_in):
    nb = _N // _R
    row = pl.BlockSpec((_R, _D), lambda i: (i, 0))
    full = pl.BlockSpec((_N, _D), lambda i: (0, 0))
    return pl.pallas_call(
        _layer_body,
        grid=(nb,),
        in_specs=[pl.BlockSpec((_R, _N), lambda i: (i, 0)), full, full, row,
                  row],
        out_specs=[row, row, row, row],
        out_shape=[jax.ShapeDtypeStruct((_N, _D), jnp.float32)] * 4,
        scratch_shapes=[pltpu.VMEM((3 * _D, _D), jnp.float32)],
        compiler_params=pltpu.CompilerParams(
            dimension_semantics=("arbitrary",),
        ),
    )(adj, lat, aa, aa, s_in)


def kernel(adj, keepRate, iEmbeds, dEmbeds, fEmbeds, iHyper, dHyper, fHyper):
    embeds = jnp.concatenate([iEmbeds, dEmbeds, fEmbeds], axis=0)
    aa = _proj(iEmbeds, dEmbeds, fEmbeds, iHyper, dHyper, fHyper)

    tem1, hyp1, lat1, sum1 = _layer(adj, embeds, aa, embeds)
    tem2, hyp2, lat2, out = _layer(adj, lat1, aa, sum1)

    return (out, tem1, tem2, hyp1, hyp2)
